# baseline (speedup 1.0000x reference)
import functools

import jax
import jax.numpy as jnp
from jax import lax
from jax.experimental import pallas as pl
from jax.experimental.pallas import tpu as pltpu
from jax.experimental.pallas import tpu_sc as plsc

VOCAB = 1000001
EMBED_D = 32
BATCH = 4096
HIST = 200

NUM_CORES = 2
NUM_SUBCORES = 16
NW = NUM_CORES * NUM_SUBCORES
B_TOT = BATCH * HIST
B_PER_W = B_TOT // NW
G = 128
K = 8
STEP = G * K
N_STEPS = B_PER_W // STEP
N_PAIRS = (N_STEPS - 1) // 2
GS = G + 1
VROWS = 1000448


def _make_gather():
    mesh = plsc.VectorSubcoreMesh(
        core_axis_name="c", subcore_axis_name="s",
        num_cores=NUM_CORES, num_subcores=NUM_SUBCORES)

    @functools.partial(
        pl.kernel,
        out_type=jax.ShapeDtypeStruct((HIST, 4, BATCH // 128, 8, 128),
                                      jnp.float32),
        mesh=mesh,
        scratch_types=[
            pltpu.VMEM((B_PER_W,), jnp.int32),
            pltpu.VMEM((STEP, EMBED_D), jnp.float32),
            pltpu.VMEM((STEP, EMBED_D), jnp.float32),
            pltpu.VMEM((4, K, 8, GS), jnp.float32),
            pltpu.SemaphoreType.DMA,
            pltpu.SemaphoreType.DMA,
            pltpu.SemaphoreType.DMA,
        ],
        compiler_params=pltpu.CompilerParams(use_tc_tiling_on_sc=False,
                                             needs_layout_passes=False),
    )
    def k(table_hbm, idx_hbm, out_hbm, idx_v, r0_v, r1_v, t_v,
          gsem0, gsem1, osem):
        wid = lax.axis_index("s") * NUM_CORES + lax.axis_index("c")
        base = wid * B_PER_W
        pltpu.sync_copy(idx_hbm.at[pl.ds(base, B_PER_W)], idx_v)
        iota = lax.iota(jnp.int32, 16)
        p0_lo = lax.shift_right_logical(iota, 3)
        p0_hi = p0_lo + 2
        p2 = lax.bitwise_and(iota, 7)

        def fire(t, r_v, sem):
            for j in range(K):
                pltpu.async_copy(
                    table_hbm.at[idx_v.at[pl.ds(t * STEP + j * G, G)]],
                    r_v.at[pl.ds(j * G, G)], sem)

        def drain(r_v, sem):
            for j in range(K):
                pltpu.make_async_copy(
                    table_hbm.at[idx_v.at[pl.ds(j * G, G)]],
                    r_v.at[pl.ds(j * G, G)], sem).wait()

        def out_slice(t):
            g = base + t * STEP
            l = g // BATCH
            cb = (g % BATCH) // G
            return out_hbm.at[l, :, pl.ds(cb, K), :, :]

        def transpose_and_put(t, r_v):
            for c in range(K):
                cv = jnp.full((16,), c, dtype=jnp.int32)

                def _j(j, c=c, cv=cv):
                    jv = jnp.full((16,), j, dtype=jnp.int32)
                    row = c * G + j
                    vals_lo = r_v[row, pl.ds(0, 16)]
                    vals_hi = r_v[row, pl.ds(16, 16)]
                    plsc.store_scatter(t_v, [p0_lo, cv, p2, jv], vals_lo)
                    plsc.store_scatter(t_v, [p0_hi, cv, p2, jv], vals_hi)

                pl.loop(0, G, unroll=4)(_j)

            pltpu.async_copy(t_v.at[:, :, :, pl.ds(0, G)], out_slice(t), osem)

        def wait_out():
            pltpu.make_async_copy(t_v.at[:, :, :, pl.ds(0, G)], out_slice(0),
                                  osem).wait()

        pltpu.async_copy(t_v.at[:, :, :, pl.ds(0, G)], out_slice(0), osem)
        fire(0, r0_v, gsem0)

        @pl.loop(0, N_PAIRS)
        def _pair(u):
            t0 = u * 2
            fire(t0 + 1, r1_v, gsem1)
            drain(r0_v, gsem0)
            wait_out()
            transpose_and_put(t0, r0_v)
            fire(t0 + 2, r0_v, gsem0)
            drain(r1_v, gsem1)
            wait_out()
            transpose_and_put(t0 + 1, r1_v)

        drain(r0_v, gsem0)
        wait_out()
        transpose_and_put(N_STEPS - 1, r0_v)
        wait_out()

    return k


_gather = _make_gather()

TCB = 1024
TC_GRID = (VOCAB + TCB - 1) // TCB
VPAD = TC_GRID * TCB


def _tc_transpose_body(in_ref, out_ref):
    y = in_ref[...].T
    z = y.reshape(TCB // 4, 4, EMBED_D)
    parts = [z[:, k, :] for k in range(4)]
    out_ref[...] = jnp.concatenate(parts, axis=1)


def _table_rowmajor(tableT):
    return pl.pallas_call(
        _tc_transpose_body,
        grid=(TC_GRID,),
        in_specs=[pl.BlockSpec((EMBED_D, TCB), lambda i: (0, i))],
        out_specs=pl.BlockSpec((TCB * EMBED_D // 128, 128), lambda i: (i, 0)),
        out_shape=jax.ShapeDtypeStruct((VPAD * EMBED_D // 128, 128),
                                       jnp.float32),
    )(tableT)


def kernel(inputs, table):
    idx = inputs.T.astype(jnp.int32).reshape(B_TOT)
    table_lin = _table_rowmajor(table.T)
    x = _gather(table_lin.reshape(VPAD, EMBED_D), idx)
    return x.transpose(0, 2, 4, 1, 3).reshape(HIST, BATCH, EMBED_D)


# speedup vs baseline: 1.3329x; 1.3329x over previous
import functools

import jax
import jax.numpy as jnp
from jax import lax
from jax.experimental import pallas as pl
from jax.experimental.pallas import tpu as pltpu
from jax.experimental.pallas import tpu_sc as plsc

VOCAB = 1000001
EMBED_D = 32
BATCH = 4096
HIST = 200

NUM_CORES = 2
NUM_SUBCORES = 16
NW = NUM_CORES * NUM_SUBCORES
B_TOT = BATCH * HIST
B_PER_W = B_TOT // NW
G = 128
K = 8
STEP = G * K
N_STEPS = B_PER_W // STEP
N_PAIRS = (N_STEPS - 1) // 2
GS = G + 1


def _make_gather():
    mesh = plsc.VectorSubcoreMesh(
        core_axis_name="c", subcore_axis_name="s",
        num_cores=NUM_CORES, num_subcores=NUM_SUBCORES)

    @functools.partial(
        pl.kernel,
        out_type=jax.ShapeDtypeStruct((HIST, 4, BATCH // 128, 8, 128),
                                      jnp.float32),
        mesh=mesh,
        scratch_types=[
            pltpu.VMEM((B_PER_W,), jnp.int32),
            pltpu.VMEM((STEP, EMBED_D), jnp.float32),
            pltpu.VMEM((STEP, EMBED_D), jnp.float32),
            pltpu.VMEM((4, K, 8, GS), jnp.float32),
            pltpu.SemaphoreType.DMA,
            pltpu.SemaphoreType.DMA,
            pltpu.SemaphoreType.DMA,
        ],
        compiler_params=pltpu.CompilerParams(use_tc_tiling_on_sc=False,
                                             needs_layout_passes=False),
    )
    def k(table_hbm, idx_hbm, out_hbm, idx_v, r0_v, r1_v, t_v,
          gsem0, gsem1, osem):
        wid = lax.axis_index("s") * NUM_CORES + lax.axis_index("c")
        base = wid * B_PER_W
        pltpu.sync_copy(idx_hbm.at[pl.ds(base, B_PER_W)], idx_v)
        iota = lax.iota(jnp.int32, 16)
        p0_lo = lax.shift_right_logical(iota, 3)
        p0_hi = p0_lo + 2
        p2 = lax.bitwise_and(iota, 7)

        def fire(t, r_v, sem):
            for j in range(K):
                pltpu.async_copy(
                    table_hbm.at[idx_v.at[pl.ds(t * STEP + j * G, G)]],
                    r_v.at[pl.ds(j * G, G)], sem)

        def drain(r_v, sem):
            for j in range(K):
                pltpu.make_async_copy(
                    table_hbm.at[idx_v.at[pl.ds(j * G, G)]],
                    r_v.at[pl.ds(j * G, G)], sem).wait()

        def out_slice(t):
            g = base + t * STEP
            l = g // BATCH
            cb = (g % BATCH) // G
            return out_hbm.at[l, :, pl.ds(cb, K), :, :]

        def transpose_and_put(t, r_v):
            for c in range(K):
                cv = jnp.full((16,), c, dtype=jnp.int32)

                def _j(j, c=c, cv=cv):
                    jv = jnp.full((16,), j, dtype=jnp.int32)
                    row = c * G + j
                    vals_lo = r_v[row, pl.ds(0, 16)]
                    vals_hi = r_v[row, pl.ds(16, 16)]
                    plsc.store_scatter(t_v, [p0_lo, cv, p2, jv], vals_lo)
                    plsc.store_scatter(t_v, [p0_hi, cv, p2, jv], vals_hi)

                pl.loop(0, G, unroll=4)(_j)

            pltpu.async_copy(t_v.at[:, :, :, pl.ds(0, G)], out_slice(t), osem)

        def wait_out():
            pltpu.make_async_copy(t_v.at[:, :, :, pl.ds(0, G)], out_slice(0),
                                  osem).wait()

        pltpu.async_copy(t_v.at[:, :, :, pl.ds(0, G)], out_slice(0), osem)
        fire(0, r0_v, gsem0)

        @pl.loop(0, N_PAIRS)
        def _pair(u):
            t0 = u * 2
            fire(t0 + 1, r1_v, gsem1)
            drain(r0_v, gsem0)
            wait_out()
            transpose_and_put(t0, r0_v)
            fire(t0 + 2, r0_v, gsem0)
            drain(r1_v, gsem1)
            wait_out()
            transpose_and_put(t0 + 1, r1_v)

        drain(r0_v, gsem0)
        wait_out()
        transpose_and_put(N_STEPS - 1, r0_v)
        wait_out()

    return k


_gather = _make_gather()

def kernel(inputs, table):
    idx = inputs.T.astype(jnp.int32).reshape(B_TOT)
    x = _gather(table, idx)
    return x.transpose(0, 2, 4, 1, 3).reshape(HIST, BATCH, EMBED_D)


# speedup vs baseline: 1.3555x; 1.0170x over previous
import functools

import jax
import jax.numpy as jnp
from jax import lax
from jax.experimental import pallas as pl
from jax.experimental.pallas import tpu as pltpu
from jax.experimental.pallas import tpu_sc as plsc

VOCAB = 1000001
EMBED_D = 32
BATCH = 4096
HIST = 200

NUM_CORES = 2
NUM_SUBCORES = 16
NW = NUM_CORES * NUM_SUBCORES
B_TOT = BATCH * HIST
B_PER_W = B_TOT // NW
G = 128
K = 8
STEP = G * K
N_STEPS = B_PER_W // STEP
N_PAIRS = (N_STEPS - 1) // 2
GS = G + 1


def _make_gather():
    mesh = plsc.VectorSubcoreMesh(
        core_axis_name="c", subcore_axis_name="s",
        num_cores=NUM_CORES, num_subcores=NUM_SUBCORES)

    @functools.partial(
        pl.kernel,
        out_type=jax.ShapeDtypeStruct((HIST, 4, BATCH // 128, 8, 128),
                                      jnp.float32),
        mesh=mesh,
        scratch_types=[
            pltpu.VMEM((B_PER_W,), jnp.int32),
            pltpu.VMEM((STEP, EMBED_D), jnp.float32),
            pltpu.VMEM((STEP, EMBED_D), jnp.float32),
            pltpu.VMEM((4, K, 8, GS), jnp.float32),
            pltpu.SemaphoreType.DMA,
            pltpu.SemaphoreType.DMA,
            pltpu.SemaphoreType.DMA,
        ],
        compiler_params=pltpu.CompilerParams(use_tc_tiling_on_sc=False,
                                             needs_layout_passes=False),
    )
    def k(table_hbm, idx_hbm, out_hbm, idx_v, r0_v, r1_v, t_v,
          gsem0, gsem1, osem):
        wid = lax.axis_index("s") * NUM_CORES + lax.axis_index("c")
        base = wid * B_PER_W
        pltpu.sync_copy(idx_hbm.at[pl.ds(base, B_PER_W)], idx_v)
        iota = lax.iota(jnp.int32, 16)
        p0_lo = lax.shift_right_logical(iota, 3)
        p0_hi = p0_lo + 2
        p2 = lax.bitwise_and(iota, 7)

        def fire(t, r_v, sem):
            for j in range(K):
                pltpu.async_copy(
                    table_hbm.at[idx_v.at[pl.ds(t * STEP + j * G, G)]],
                    r_v.at[pl.ds(j * G, G)], sem)

        def drain(r_v, sem):
            for j in range(K):
                pltpu.make_async_copy(
                    table_hbm.at[idx_v.at[pl.ds(j * G, G)]],
                    r_v.at[pl.ds(j * G, G)], sem).wait()

        def out_slice(t):
            g = base + t * STEP
            l = g // BATCH
            cb = (g % BATCH) // G
            return out_hbm.at[l, :, pl.ds(cb, K), :, :]

        def transpose_and_put(t, r_v):
            for c in range(K):
                cv = jnp.full((16,), c, dtype=jnp.int32)

                def _j(j, c=c, cv=cv):
                    jv = jnp.full((16,), j, dtype=jnp.int32)
                    row = c * G + j
                    vals_lo = r_v[row, pl.ds(0, 16)]
                    vals_hi = r_v[row, pl.ds(16, 16)]
                    plsc.store_scatter(t_v, [p0_lo, cv, p2, jv], vals_lo)
                    plsc.store_scatter(t_v, [p0_hi, cv, p2, jv], vals_hi)

                pl.loop(0, G, unroll=4)(_j)

            pltpu.async_copy(t_v.at[:, :, :, pl.ds(0, G)], out_slice(t), osem)

        def wait_out():
            pltpu.make_async_copy(t_v.at[:, :, :, pl.ds(0, G)], out_slice(0),
                                  osem).wait()

        pltpu.async_copy(t_v.at[:, :, :, pl.ds(0, G)], out_slice(0), osem)
        fire(0, r0_v, gsem0)

        @pl.loop(0, N_PAIRS)
        def _pair(u):
            t0 = u * 2
            fire(t0 + 1, r1_v, gsem1)
            drain(r0_v, gsem0)
            wait_out()
            transpose_and_put(t0, r0_v)
            fire(t0 + 2, r0_v, gsem0)
            drain(r1_v, gsem1)
            wait_out()
            transpose_and_put(t0 + 1, r1_v)

        drain(r0_v, gsem0)
        wait_out()
        transpose_and_put(N_STEPS - 1, r0_v)
        wait_out()

    return k


_gather = _make_gather()

def kernel(inputs, table):
    idx = (inputs.T.astype(jnp.int32) * 4).reshape(B_TOT)
    tp = jnp.pad(table, ((0, 0), (0, 128 - EMBED_D)))
    x = _gather(tp.reshape(VOCAB * 4, EMBED_D), idx)
    return x.transpose(0, 2, 4, 1, 3).reshape(HIST, BATCH, EMBED_D)
